# Initial kernel scaffold; baseline (speedup 1.0000x reference)
import jax
import jax.numpy as jnp
from jax import lax
from jax.experimental import pallas as pl
from jax.experimental.pallas import tpu as pltpu
from jax.experimental.pallas import tpu_sc as plsc

BATCH = 16384
SEQ = 150
DIM = 20
N_TOTAL = BATCH * SEQ

NUM_CORES = 2
NUM_SUBCORES = 16
NW = NUM_CORES * NUM_SUBCORES
PER_W = N_TOTAL // NW
CHUNK = 2400
N_CHUNKS = PER_W // CHUNK


def _emb_body(idx_hbm, table_hbm, out_hbm, idx_v, rows_v, sem):
    wid = lax.axis_index("s") * NUM_CORES + lax.axis_index("c")
    base = wid * PER_W

    def step(g, carry):
        off = base + g * CHUNK
        pltpu.sync_copy(idx_hbm.at[pl.ds(off, CHUNK)], idx_v)
        pltpu.async_copy(table_hbm.at[idx_v], rows_v, sem).wait()
        pltpu.sync_copy(rows_v, out_hbm.at[pl.ds(off, CHUNK)])
        return carry

    lax.fori_loop(0, N_CHUNKS, step, 0)


def kernel(pad_indexes, embedding_table):
    idx = pad_indexes.reshape(N_TOTAL).astype(jnp.int32)
    mesh = plsc.VectorSubcoreMesh(
        core_axis_name="c", subcore_axis_name="s",
        num_cores=NUM_CORES, num_subcores=NUM_SUBCORES,
    )
    out = pl.kernel(
        _emb_body,
        out_type=jax.ShapeDtypeStruct((N_TOTAL, DIM), jnp.float32),
        mesh=mesh,
        scratch_types=[
            pltpu.VMEM((CHUNK,), jnp.int32),
            pltpu.VMEM((CHUNK, DIM), jnp.float32),
            pltpu.SemaphoreType.DMA,
        ],
    )(idx, embedding_table)
    return out.reshape(BATCH, SEQ, DIM)


# speedup vs baseline: 2.2734x; 2.2734x over previous
import jax
import jax.numpy as jnp
from jax import lax
from jax.experimental import pallas as pl
from jax.experimental.pallas import tpu as pltpu
from jax.experimental.pallas import tpu_sc as plsc

BATCH = 16384
SEQ = 150
DIM = 20
N_ROWS = 155
N_TOTAL = BATCH * SEQ

NUM_CORES = 2
NUM_SUBCORES = 16
NW = NUM_CORES * NUM_SUBCORES
PER_W = N_TOTAL // NW
GRP = 128
NSUB = 10
CHUNK = GRP * NSUB
N_CHUNKS = PER_W // CHUNK


def _emb_body(idx_hbm, table_hbm, out_hbm, table_s, idx_v, rows_v, sem):
    sid = lax.axis_index("s")
    wid = sid * NUM_CORES + lax.axis_index("c")
    base = wid * PER_W

    @pl.when(sid == 0)
    def _stage_table():
        pltpu.sync_copy(table_hbm, table_s)

    plsc.subcore_barrier()

    def step(g, carry):
        off = base + g * CHUNK
        pltpu.sync_copy(idx_hbm.at[pl.ds(off // GRP, NSUB)], idx_v)
        for j in range(NSUB):
            pltpu.async_copy(
                table_s.at[idx_v.at[j]],
                rows_v.at[pl.ds(j * GRP, GRP)],
                sem,
            )
        for j in range(NSUB):
            pltpu.make_async_copy(
                table_s.at[idx_v.at[j]],
                rows_v.at[pl.ds(j * GRP, GRP)],
                sem,
            ).wait()
        pltpu.sync_copy(rows_v, out_hbm.at[pl.ds(off, CHUNK)])
        return carry

    lax.fori_loop(0, N_CHUNKS, step, 0)


def kernel(pad_indexes, embedding_table):
    idx = pad_indexes.reshape(N_TOTAL // GRP, GRP).astype(jnp.int32)
    mesh = plsc.VectorSubcoreMesh(
        core_axis_name="c", subcore_axis_name="s",
        num_cores=NUM_CORES, num_subcores=NUM_SUBCORES,
    )
    out = pl.kernel(
        _emb_body,
        out_type=jax.ShapeDtypeStruct((N_TOTAL, DIM), jnp.float32),
        mesh=mesh,
        scratch_types=[
            pltpu.VMEM_SHARED((N_ROWS, DIM), jnp.float32),
            pltpu.VMEM((NSUB, GRP), jnp.int32),
            pltpu.VMEM((CHUNK, DIM), jnp.float32),
            pltpu.SemaphoreType.DMA,
        ],
        compiler_params=pltpu.CompilerParams(use_tc_tiling_on_sc=False),
    )(idx, embedding_table)
    return out.reshape(BATCH, SEQ, DIM)


# speedup vs baseline: 6.1765x; 2.7168x over previous
import jax
import jax.numpy as jnp
from jax import lax
from jax.experimental import pallas as pl
from jax.experimental.pallas import tpu as pltpu
from jax.experimental.pallas import tpu_sc as plsc

BATCH = 16384
SEQ = 150
DIM = 20
N_ROWS = 155

NUM_CORES = 2
NUM_SUBCORES = 16
NW = NUM_CORES * NUM_SUBCORES
B_PER_W = BATCH // NW
KB_I = 8
N_STEPS = B_PER_W // KB_I

SROWS = 152
SCOLS = 128
GROUPS = (0, 16, 32, 48, 64, 80, 96, 112, 128, 134)


def _emb_body(idx_hbm, table_hbm, out_hbm, table_v, idx_v, stage, sem):
    wid = lax.axis_index("s") * NUM_CORES + lax.axis_index("c")
    base = wid * B_PER_W

    pltpu.sync_copy(table_hbm, table_v)
    lanes = lax.iota(jnp.int32, 16)

    def idx_step(g, carry):
        b0 = base + g * KB_I
        pltpu.sync_copy(idx_hbm.at[pl.ds(b0, KB_I)], idx_v)

        def batch_block(t, carry2):
            for c0 in GROUPS:
                rows = idx_v[t, pl.ds(c0, 16)] * DIM
                srow = lanes + c0
                for c in range(DIM):
                    vals = plsc.load_gather(table_v, [rows + c])
                    plsc.store_scatter(
                        stage, [srow, jnp.full((16,), c, jnp.int32)], vals)
            pltpu.sync_copy(stage, out_hbm.at[b0 + t])
            return carry2

        lax.fori_loop(0, KB_I, batch_block, 0)
        return carry

    lax.fori_loop(0, N_STEPS, idx_step, 0)


def kernel(pad_indexes, embedding_table):
    idx = pad_indexes.astype(jnp.int32)
    table_flat = embedding_table.reshape(N_ROWS * DIM)
    mesh = plsc.VectorSubcoreMesh(
        core_axis_name="c", subcore_axis_name="s",
        num_cores=NUM_CORES, num_subcores=NUM_SUBCORES,
    )
    out = pl.kernel(
        _emb_body,
        out_type=jax.ShapeDtypeStruct((BATCH, SEQ, DIM), jnp.float32),
        mesh=mesh,
        scratch_types=[
            pltpu.VMEM((N_ROWS * DIM,), jnp.float32),
            pltpu.VMEM((KB_I, SEQ), jnp.int32),
            pltpu.VMEM((SEQ, DIM), jnp.float32),
            pltpu.SemaphoreType.DMA,
        ],
        compiler_params=pltpu.CompilerParams(needs_layout_passes=False),
    )(idx, table_flat)
    return out


# speedup vs baseline: 7.3299x; 1.1867x over previous
import jax
import jax.numpy as jnp
from jax import lax
from jax.experimental import pallas as pl
from jax.experimental.pallas import tpu as pltpu
from jax.experimental.pallas import tpu_sc as plsc

BATCH = 16384
SEQ = 150
DIM = 20
N_ROWS = 155

NUM_CORES = 2
NUM_SUBCORES = 16
NW = NUM_CORES * NUM_SUBCORES
B_PER_W = BATCH // NW
KB_I = 64
N_ISTEPS = B_PER_W // KB_I
NBUF = 4
N_RINGS = KB_I // NBUF

GROUPS = (0, 16, 32, 48, 64, 80, 96, 112, 128, 134)


def _emb_body(idx_hbm, table_hbm, out_hbm,
              table_v, idx_v, s0, s1, s2, s3, m0, m1, m2, m3):
    wid = lax.axis_index("s") * NUM_CORES + lax.axis_index("c")
    base = wid * B_PER_W

    pltpu.sync_copy(table_hbm, table_v)
    lanes = lax.iota(jnp.int32, 16)
    srows = [lanes + c0 for c0 in GROUPS]
    cfull = [jnp.full((16,), c, jnp.int32) for c in range(DIM)]
    stages = (s0, s1, s2, s3)
    sems = (m0, m1, m2, m3)

    def fill(stage, t):
        for gi, c0 in enumerate(GROUPS):
            rows = idx_v[t, pl.ds(c0, 16)] * DIM
            for c in range(DIM):
                vals = plsc.load_gather(table_v, [rows + cfull[c]])
                plsc.store_scatter(stage, [srows[gi], cfull[c]], vals)

    def idx_step(g, carry):
        b0 = base + g * KB_I
        pltpu.sync_copy(idx_hbm.at[pl.ds(b0, KB_I)], idx_v)

        def ring_step(s, carry2):
            for p in range(NBUF):
                t = s * NBUF + p

                @pl.when(jnp.logical_or(g > 0, s > 0))
                def _drain():
                    pltpu.make_async_copy(
                        stages[p], out_hbm.at[base], sems[p]
                    ).wait()

                fill(stages[p], t)
                pltpu.async_copy(stages[p], out_hbm.at[b0 + t], sems[p])
            return carry2

        lax.fori_loop(0, N_RINGS, ring_step, 0)
        return carry

    lax.fori_loop(0, N_ISTEPS, idx_step, 0)

    for p in range(NBUF):
        pltpu.make_async_copy(stages[p], out_hbm.at[base], sems[p]).wait()


def kernel(pad_indexes, embedding_table):
    idx = pad_indexes.astype(jnp.int32)
    table_flat = embedding_table.reshape(N_ROWS * DIM)
    mesh = plsc.VectorSubcoreMesh(
        core_axis_name="c", subcore_axis_name="s",
        num_cores=NUM_CORES, num_subcores=NUM_SUBCORES,
    )
    out = pl.kernel(
        _emb_body,
        out_type=jax.ShapeDtypeStruct((BATCH, SEQ, DIM), jnp.float32),
        mesh=mesh,
        scratch_types=[
            pltpu.VMEM((N_ROWS * DIM,), jnp.float32),
            pltpu.VMEM((KB_I, SEQ), jnp.int32),
            pltpu.VMEM((SEQ, DIM), jnp.float32),
            pltpu.VMEM((SEQ, DIM), jnp.float32),
            pltpu.VMEM((SEQ, DIM), jnp.float32),
            pltpu.VMEM((SEQ, DIM), jnp.float32),
            pltpu.SemaphoreType.DMA,
            pltpu.SemaphoreType.DMA,
            pltpu.SemaphoreType.DMA,
            pltpu.SemaphoreType.DMA,
        ],
        compiler_params=pltpu.CompilerParams(needs_layout_passes=False),
    )(idx, table_flat)
    return out


# speedup vs baseline: 7.3389x; 1.0012x over previous
import jax
import jax.numpy as jnp
from jax import lax
from jax.experimental import pallas as pl
from jax.experimental.pallas import tpu as pltpu
from jax.experimental.pallas import tpu_sc as plsc

BATCH = 16384
SEQ = 150
DIM = 20
N_ROWS = 155

NUM_CORES = 2
NUM_SUBCORES = 16
NW = NUM_CORES * NUM_SUBCORES
B_PER_W = BATCH // NW
KB_I = 128
N_ISTEPS = B_PER_W // KB_I
NBUF = 4
N_RINGS = KB_I // NBUF

GROUPS = (0, 16, 32, 48, 64, 80, 96, 112, 128, 134)


def _emb_body(idx_hbm, table_hbm, out_hbm,
              table_v, idx_v, s0, s1, s2, s3, m0, m1, m2, m3):
    wid = lax.axis_index("s") * NUM_CORES + lax.axis_index("c")
    base = wid * B_PER_W

    pltpu.sync_copy(table_hbm, table_v)
    lanes = lax.iota(jnp.int32, 16)
    srows = [lanes + c0 for c0 in GROUPS]
    cfull = [jnp.full((16,), c, jnp.int32) for c in range(DIM)]
    stages = (s0, s1, s2, s3)
    sems = (m0, m1, m2, m3)

    def fill(stage, t):
        for gi, c0 in enumerate(GROUPS):
            rows = idx_v[t, pl.ds(c0, 16)] * DIM
            for c in range(DIM):
                vals = plsc.load_gather(table_v, [rows + cfull[c]])
                plsc.store_scatter(stage, [srows[gi], cfull[c]], vals)

    def idx_step(g, carry):
        b0 = base + g * KB_I
        pltpu.sync_copy(idx_hbm.at[pl.ds(b0, KB_I)], idx_v)

        def ring_step(s, carry2):
            for p in range(NBUF):
                t = s * NBUF + p

                @pl.when(jnp.logical_or(g > 0, s > 0))
                def _drain():
                    pltpu.make_async_copy(
                        stages[p], out_hbm.at[base], sems[p]
                    ).wait()

                fill(stages[p], t)
                pltpu.async_copy(stages[p], out_hbm.at[b0 + t], sems[p])
            return carry2

        lax.fori_loop(0, N_RINGS, ring_step, 0)
        return carry

    lax.fori_loop(0, N_ISTEPS, idx_step, 0)

    for p in range(NBUF):
        pltpu.make_async_copy(stages[p], out_hbm.at[base], sems[p]).wait()


def kernel(pad_indexes, embedding_table):
    idx = pad_indexes.astype(jnp.int32)
    table_flat = embedding_table.reshape(N_ROWS * DIM)
    mesh = plsc.VectorSubcoreMesh(
        core_axis_name="c", subcore_axis_name="s",
        num_cores=NUM_CORES, num_subcores=NUM_SUBCORES,
    )
    out = pl.kernel(
        _emb_body,
        out_type=jax.ShapeDtypeStruct((BATCH, SEQ, DIM), jnp.float32),
        mesh=mesh,
        scratch_types=[
            pltpu.VMEM((N_ROWS * DIM,), jnp.float32),
            pltpu.VMEM((KB_I, SEQ), jnp.int32),
            pltpu.VMEM((SEQ, DIM), jnp.float32),
            pltpu.VMEM((SEQ, DIM), jnp.float32),
            pltpu.VMEM((SEQ, DIM), jnp.float32),
            pltpu.VMEM((SEQ, DIM), jnp.float32),
            pltpu.SemaphoreType.DMA,
            pltpu.SemaphoreType.DMA,
            pltpu.SemaphoreType.DMA,
            pltpu.SemaphoreType.DMA,
        ],
        compiler_params=pltpu.CompilerParams(needs_layout_passes=False),
    )(idx, table_flat)
    return out


# speedup vs baseline: 7.3477x; 1.0012x over previous
import jax
import jax.numpy as jnp
from jax import lax
from jax.experimental import pallas as pl
from jax.experimental.pallas import tpu as pltpu
from jax.experimental.pallas import tpu_sc as plsc

BATCH = 16384
SEQ = 150
DIM = 20
N_ROWS = 155

NUM_CORES = 2
NUM_SUBCORES = 16
NW = NUM_CORES * NUM_SUBCORES
B_PER_W = BATCH // NW
KB_I = 128
N_ISTEPS = B_PER_W // KB_I
NBUF = 2
RPB = 2
N_RINGS = KB_I // (NBUF * RPB)

GROUPS = (0, 16, 32, 48, 64, 80, 96, 112, 128, 134)


def _emb_body(idx_hbm, table_hbm, out_hbm,
              table_v, idx_v, s0, s1, m0, m1):
    wid = lax.axis_index("s") * NUM_CORES + lax.axis_index("c")
    base = wid * B_PER_W

    pltpu.sync_copy(table_hbm, table_v)
    lanes = lax.iota(jnp.int32, 16)
    srows = [lanes + c0 for c0 in GROUPS]
    cfull = [jnp.full((16,), c, jnp.int32) for c in range(DIM)]
    stages = (s0, s1)
    sems = (m0, m1)

    def fill(stage, t):
        for gi, c0 in enumerate(GROUPS):
            rows = idx_v[t, pl.ds(c0, 16)] * DIM
            for c in range(DIM):
                vals = plsc.load_gather(table_v, [rows + cfull[c]])
                plsc.store_scatter(stage, [srows[gi], cfull[c]], vals)

    def idx_step(g, carry):
        b0 = base + g * KB_I
        pltpu.sync_copy(idx_hbm.at[pl.ds(b0, KB_I)], idx_v)

        def ring_step(s, carry2):
            for p in range(NBUF):
                t = (s * NBUF + p) * RPB

                @pl.when(jnp.logical_or(g > 0, s > 0))
                def _drain():
                    pltpu.make_async_copy(
                        stages[p], out_hbm.at[pl.ds(base, RPB)], sems[p]
                    ).wait()

                for r in range(RPB):
                    fill(stages[p].at[r], t + r)
                pltpu.async_copy(
                    stages[p], out_hbm.at[pl.ds(b0 + t, RPB)], sems[p])
            return carry2

        lax.fori_loop(0, N_RINGS, ring_step, 0)
        return carry

    lax.fori_loop(0, N_ISTEPS, idx_step, 0)

    for p in range(NBUF):
        pltpu.make_async_copy(
            stages[p], out_hbm.at[pl.ds(base, RPB)], sems[p]).wait()


def kernel(pad_indexes, embedding_table):
    idx = pad_indexes.astype(jnp.int32)
    table_flat = embedding_table.reshape(N_ROWS * DIM)
    mesh = plsc.VectorSubcoreMesh(
        core_axis_name="c", subcore_axis_name="s",
        num_cores=NUM_CORES, num_subcores=NUM_SUBCORES,
    )
    out = pl.kernel(
        _emb_body,
        out_type=jax.ShapeDtypeStruct((BATCH, SEQ, DIM), jnp.float32),
        mesh=mesh,
        scratch_types=[
            pltpu.VMEM((N_ROWS * DIM,), jnp.float32),
            pltpu.VMEM((KB_I, SEQ), jnp.int32),
            pltpu.VMEM((RPB, SEQ, DIM), jnp.float32),
            pltpu.VMEM((RPB, SEQ, DIM), jnp.float32),
            pltpu.SemaphoreType.DMA,
            pltpu.SemaphoreType.DMA,
        ],
        compiler_params=pltpu.CompilerParams(needs_layout_passes=False),
    )(idx, table_flat)
    return out
